# speedup vs baseline: 5.0027x; 5.0027x over previous
import jax
import jax.numpy as jnp
from jax.experimental import pallas as pl
from jax.experimental.pallas import tpu as pltpu

NPOS = 8192
NPRIM = 1024
NORB = 256
BN = 512


def _ipow(d, l):
    return jnp.where(l == 0, 1.0, jnp.where(l == 1, d, d * d))


def _basis_block(pos_ref, cn_ref, centerT_ref, alpha_ref, lmnT_ref, oi_ref,
                 out_ref):
    p = pos_ref[...]
    x = p[:, 0:1]
    y = p[:, 1:2]
    z = p[:, 2:3]

    cx = centerT_ref[0:1, :]
    cy = centerT_ref[1:2, :]
    cz = centerT_ref[2:3, :]

    dx = x - cx
    dy = y - cy
    dz = z - cz
    r2 = dx * dx + dy * dy + dz * dz

    lx = lmnT_ref[0:1, :]
    ly = lmnT_ref[1:2, :]
    lz = lmnT_ref[2:3, :]
    ang = _ipow(dx, lx) * _ipow(dy, ly) * _ipow(dz, lz)

    prim = cn_ref[...] * ang * jnp.exp(-alpha_ref[...] * r2)

    col = jax.lax.broadcasted_iota(jnp.int32, (NORB, NPRIM), 0)
    s = (col == oi_ref[...]).astype(jnp.float32)
    out_ref[...] = jax.lax.dot_general(
        prim, s, (((1,), (1,)), ((), ())),
        preferred_element_type=jnp.float32)


@jax.jit
def kernel(pos, coefficients, center, alpha, norm, lmn, orbital_index):
    cn = (coefficients * norm).reshape(1, NPRIM)
    centerT = center.T
    lmnT = lmn.T
    alpha2 = alpha.reshape(1, NPRIM)
    oi = orbital_index.reshape(1, NPRIM)

    grid = (NPOS // BN,)
    return pl.pallas_call(
        _basis_block,
        grid=grid,
        in_specs=[
            pl.BlockSpec((BN, 3), lambda i: (i, 0)),
            pl.BlockSpec((1, NPRIM), lambda i: (0, 0)),
            pl.BlockSpec((3, NPRIM), lambda i: (0, 0)),
            pl.BlockSpec((1, NPRIM), lambda i: (0, 0)),
            pl.BlockSpec((3, NPRIM), lambda i: (0, 0)),
            pl.BlockSpec((1, NPRIM), lambda i: (0, 0)),
        ],
        out_specs=pl.BlockSpec((BN, NORB), lambda i: (i, 0)),
        out_shape=jax.ShapeDtypeStruct((NPOS, NORB), jnp.float32),
        compiler_params=pltpu.CompilerParams(
            dimension_semantics=("parallel",)),
    )(pos, cn, centerT, alpha2, lmnT, oi)
